# baseline (speedup 1.0000x reference)
import functools

import jax
import jax.numpy as jnp
from jax import lax
from jax.experimental import pallas as pl
from jax.experimental.pallas import tpu as pltpu
from jax.experimental.pallas import tpu_sc as plsc

B, A, C = 32, 16384, 81
N = B * A
ROWS = 16384
SPB = A // ROWS
STEPS = N // ROWS
CH = 4096
NW = 32


def _ce_kernel(clsa_ref, clsb_ref, tgta_ref, tgtb_ref, out_ref, acc_ref):
    step = pl.program_id(0)

    @pl.when(step == 0)
    def _init():
        acc_ref[0] = 0.0
        acc_ref[1] = 0.0

    crow = jax.lax.broadcasted_iota(jnp.int32, (C, ROWS), 0)
    ce = 0.0
    npos = 0.0
    for cref, tref in ((clsa_ref, tgta_ref), (clsb_ref, tgtb_ref)):
        x = cref[0]
        xt = jnp.transpose(x)
        tv = tref[0]
        e = jnp.exp(xt)
        s = jnp.sum(e, axis=0, keepdims=True)
        ce += jnp.sum(jnp.log(s))
        ce -= jnp.sum(jnp.where(crow == tv, xt, 0.0))
        npos += jnp.sum((tv != 0).astype(jnp.float32))

    acc_ref[0] += ce
    acc_ref[1] += npos

    @pl.when(step == STEPS // 2 - 1)
    def _fini():
        out_ref[0] = acc_ref[0]
        out_ref[1] = acc_ref[1]


def _sc_loc_kernel(locp_hbm, loct_hbm, ct_hbm, out_hbm, lp_v, lt_v, ct_v,
                   acc_v):
    wid = lax.axis_index("s") * 2 + lax.axis_index("c")

    io = lax.iota(jnp.int32, 16)
    four = jnp.full((16,), 4, jnp.int32)
    idiv4 = lax.shift_right_logical(io, jnp.full((16,), 2, jnp.int32))
    imod4 = lax.bitwise_and(io, jnp.full((16,), 3, jnp.int32))
    zero = jnp.zeros((16,), jnp.float32)
    one = jnp.full((16,), 1.0, jnp.float32)
    half = jnp.full((16,), 0.5, jnp.float32)
    izero = jnp.zeros((16,), jnp.int32)
    ifour = jnp.full((16,), 4, jnp.int32)
    two = jnp.full((16,), 2, jnp.int32)
    ten = jnp.full((16,), 10.0, jnp.float32)
    five = jnp.full((16,), 5.0, jnp.float32)
    inv_std = jnp.where(imod4 < two, ten, five)

    def chunk_body(c, acc):
        c0 = c * CH
        pltpu.sync_copy(locp_hbm.at[wid, pl.ds(c0 * 4, CH * 4)], lp_v)
        pltpu.sync_copy(loct_hbm.at[wid, pl.ds(c0 * 4, CH * 4)], lt_v)
        pltpu.sync_copy(ct_hbm.at[wid, pl.ds(c0, CH)], ct_v)

        def vec_body(g, a):
            t16 = ct_v[pl.ds(g * 16, 16)]
            posf = jnp.where(t16 != izero, one, zero)
            for j in range(4):
                lp16 = lp_v[pl.ds((g * 4 + j) * 16, 16)]
                lt16 = lt_v[pl.ds((g * 4 + j) * 16, 16)]
                pexp = posf.at[idiv4 + 4 * j].get(mode="promise_in_bounds")
                d = lp16 - lt16 * inv_std
                ad = jnp.abs(d)
                sl1 = jnp.where(ad < one, half * ad * ad, ad - half)
                a = a + sl1 * pexp
            return a

        return lax.fori_loop(0, CH // 16, vec_body, acc)

    acc = lax.fori_loop(0, A // CH, chunk_body, zero)
    acc_v[...] = acc
    pltpu.sync_copy(acc_v, out_hbm.at[wid])


_sc_loc = functools.partial(
    pl.kernel,
    mesh=plsc.VectorSubcoreMesh(core_axis_name="c", subcore_axis_name="s"),
    out_type=jax.ShapeDtypeStruct((NW, 16), jnp.float32),
    scratch_types=[
        pltpu.VMEM((CH * 4,), jnp.float32),
        pltpu.VMEM((CH * 4,), jnp.float32),
        pltpu.VMEM((CH,), jnp.int32),
        pltpu.VMEM((16,), jnp.float32),
    ],
)(_sc_loc_kernel)


@jax.jit
def kernel(loc_p, cls_p, loc_t, cls_t):
    ct = cls_t.astype(jnp.int32)
    tgtv = ct.reshape(STEPS, 1, ROWS)

    hb = B // 2
    ce_np = pl.pallas_call(
        _ce_kernel,
        grid=(STEPS // 2,),
        in_specs=[
            pl.BlockSpec((1, ROWS, C), lambda i: (i, 0, 0)),
            pl.BlockSpec((1, ROWS, C), lambda i: (i + hb, 0, 0)),
            pl.BlockSpec((1, 1, ROWS), lambda i: (i, 0, 0)),
            pl.BlockSpec((1, 1, ROWS), lambda i: (i + hb, 0, 0)),
        ],
        out_specs=pl.BlockSpec(memory_space=pltpu.SMEM),
        out_shape=jax.ShapeDtypeStruct((2,), jnp.float32),
        scratch_shapes=[pltpu.SMEM((2,), jnp.float32)],
    )(cls_p, cls_p, tgtv, tgtv)

    loc_parts = _sc_loc(loc_p.reshape(B, A * 4), loc_t.reshape(B, A * 4), ct)
    return (ce_np[0] + jnp.sum(loc_parts)) / ce_np[1]
